# baseline (speedup 1.0000x reference)
import functools
import math

import jax
import jax.numpy as jnp
from jax import lax
from jax.experimental import pallas as pl
from jax.experimental.pallas import tpu as pltpu

_LOG2E = 1.4426950408889634


def _prep_kernel(zi_ref, zj_ref, reps_ref, kti_ref, ktj_ref, pos_ref, sd_ref,
                 *, scale2, inv_t):
    zi = zi_ref[...]
    zj = zj_ref[...]
    zi_n = zi * lax.rsqrt(jnp.maximum(jnp.sum(zi * zi, axis=-1, keepdims=True),
                                      1e-24))
    zj_n = zj * lax.rsqrt(jnp.maximum(jnp.sum(zj * zj, axis=-1, keepdims=True),
                                      1e-24))
    pos_ref[...] = jnp.float32(inv_t) * jnp.sum(zi_n * zj_n, axis=-1,
                                                keepdims=True)
    a = (zi_n * jnp.float32(scale2)).astype(jnp.bfloat16)
    b = (zj_n * jnp.float32(scale2)).astype(jnp.bfloat16)
    reps_ref[0] = a
    reps_ref[1] = b
    kti_ref[...] = a.T
    ktj_ref[...] = b.T
    af = a.astype(jnp.float32)
    bf = b.astype(jnp.float32)
    sd_ref[0] = jnp.sum(af * af, axis=-1, keepdims=True)
    sd_ref[1] = jnp.sum(bf * bf, axis=-1, keepdims=True)


def _lse_kernel(q_ref, kti_ref, ktj_ref, sd_ref, lse_ref, *, sub, acc_w):
    q = q_ref[...]
    part0 = None
    part1 = None
    for kt_ref in (kti_ref, ktj_ref):
        width = kt_ref.shape[-1]
        for c in range(width // sub):
            s = jnp.dot(q, kt_ref[:, c * sub:(c + 1) * sub],
                        preferred_element_type=jnp.float32)
            p = jnp.exp2(s)
            for j in range(sub // acc_w):
                chunk = p[:, j * acc_w:(j + 1) * acc_w]
                if j % 2 == 0:
                    part0 = chunk if part0 is None else part0 + chunk
                else:
                    part1 = chunk if part1 is None else part1 + chunk
    part = part0 if part1 is None else part0 + part1
    s_row = jnp.sum(part, axis=-1, keepdims=True)
    lse_ref[...] = jnp.log(s_row - jnp.exp2(sd_ref[...]))


def _round_up(x, mult):
    return (x + mult - 1) // mult * mult


def _pick_block(total, candidates):
    for c in candidates:
        if c <= total and total % c == 0:
            return c
    return total


def kernel(z_i, z_j, temperature=0.5):
    assert z_i.shape == z_j.shape and z_i.ndim == 2
    n, d = z_i.shape
    m = 2 * n
    inv_t = 1.0 / float(temperature)
    scale2 = math.sqrt(inv_t * _LOG2E)

    d_pad = max(128, _round_up(d, 128))
    if d_pad != d:
        z_i = jnp.pad(z_i, ((0, 0), (0, d_pad - d)))
        z_j = jnp.pad(z_j, ((0, 0), (0, d_pad - d)))

    bn = _pick_block(n, (256, 128, 64, 32, 16, 8))

    reps, kti, ktj, pos, sd = pl.pallas_call(
        functools.partial(_prep_kernel, scale2=scale2, inv_t=inv_t),
        grid=(n // bn,),
        in_specs=[pl.BlockSpec((bn, d_pad), lambda i: (i, 0)),
                  pl.BlockSpec((bn, d_pad), lambda i: (i, 0))],
        out_specs=(pl.BlockSpec((2, bn, d_pad), lambda i: (0, i, 0)),
                   pl.BlockSpec((d_pad, bn), lambda i: (0, i)),
                   pl.BlockSpec((d_pad, bn), lambda i: (0, i)),
                   pl.BlockSpec((bn, 1), lambda i: (i, 0)),
                   pl.BlockSpec((2, bn, 1), lambda i: (0, i, 0))),
        out_shape=(jax.ShapeDtypeStruct((2, n, d_pad), jnp.bfloat16),
                   jax.ShapeDtypeStruct((d_pad, n), jnp.bfloat16),
                   jax.ShapeDtypeStruct((d_pad, n), jnp.bfloat16),
                   jax.ShapeDtypeStruct((n, 1), jnp.float32),
                   jax.ShapeDtypeStruct((2, n, 1), jnp.float32)),
        compiler_params=pltpu.CompilerParams(
            dimension_semantics=("parallel",),
            vmem_limit_bytes=48 * 1024 * 1024),
    )(z_i, z_j)

    q = reps.reshape(m, d_pad)
    sd_m = sd.reshape(m, 1)

    bq = _pick_block(m, (1024, 512, 256, 128, 64, 32, 16, 8))
    sub = min(n, 1024)
    acc_w = 128 if sub % 128 == 0 else sub

    est2 = (2 * m * d_pad * 2
            + 2 * bq * d_pad * 2
            + 8 * bq * sub * 4)
    cost = pl.CostEstimate(flops=2 * m * m * d_pad,
                           transcendentals=m * m,
                           bytes_accessed=2 * m * d_pad * 2 + m * 4)

    lse = pl.pallas_call(
        functools.partial(_lse_kernel, sub=sub, acc_w=acc_w),
        grid=(m // bq,),
        in_specs=[pl.BlockSpec((bq, d_pad), lambda i: (i, 0)),
                  pl.BlockSpec((d_pad, n), lambda i: (0, 0)),
                  pl.BlockSpec((d_pad, n), lambda i: (0, 0)),
                  pl.BlockSpec((bq, 1), lambda i: (i, 0))],
        out_specs=pl.BlockSpec((bq, 1), lambda i: (i, 0)),
        out_shape=jax.ShapeDtypeStruct((m, 1), jnp.float32),
        compiler_params=pltpu.CompilerParams(
            dimension_semantics=("arbitrary",),
            vmem_limit_bytes=min(64 * 1024 * 1024,
                                 max(32 * 1024 * 1024, 2 * est2))),
        cost_estimate=cost,
    )(q, kti, ktj, sd_m)

    return (jnp.sum(lse) - 2.0 * jnp.sum(pos)) / jnp.float32(m)


# speedup vs baseline: 1.0149x; 1.0149x over previous
import functools
import math

import jax
import jax.numpy as jnp
from jax import lax
from jax.experimental import pallas as pl
from jax.experimental.pallas import tpu as pltpu

_LOG2E = 1.4426950408889634


def _prep_kernel(zi_ref, zj_ref, reps_ref, kti_ref, ktj_ref, pos_ref, sd_ref,
                 *, scale2, inv_t):
    zi = zi_ref[...]
    zj = zj_ref[...]
    zi_n = zi * lax.rsqrt(jnp.maximum(jnp.sum(zi * zi, axis=-1, keepdims=True),
                                      1e-24))
    zj_n = zj * lax.rsqrt(jnp.maximum(jnp.sum(zj * zj, axis=-1, keepdims=True),
                                      1e-24))
    pos_ref[...] = jnp.float32(inv_t) * jnp.sum(zi_n * zj_n, axis=-1,
                                                keepdims=True)
    a = (zi_n * jnp.float32(scale2)).astype(jnp.bfloat16)
    b = (zj_n * jnp.float32(scale2)).astype(jnp.bfloat16)
    reps_ref[0] = a
    reps_ref[1] = b
    kti_ref[...] = a.T
    ktj_ref[...] = b.T
    af = a.astype(jnp.float32)
    bf = b.astype(jnp.float32)
    sd_ref[0] = jnp.sum(af * af, axis=-1, keepdims=True)
    sd_ref[1] = jnp.sum(bf * bf, axis=-1, keepdims=True)


def _lse_kernel(q_ref, kti_ref, ktj_ref, sd_ref, lse_ref, *, sub, acc_w):
    q = q_ref[...]
    part0 = None
    part1 = None
    for kt_ref in (kti_ref, ktj_ref):
        width = kt_ref.shape[-1]
        for c in range(width // sub):
            s = jnp.dot(q, kt_ref[:, c * sub:(c + 1) * sub],
                        preferred_element_type=jnp.float32)
            p = jnp.exp2(s)
            for j in range(sub // acc_w):
                chunk = p[:, j * acc_w:(j + 1) * acc_w]
                if j % 2 == 0:
                    part0 = chunk if part0 is None else part0 + chunk
                else:
                    part1 = chunk if part1 is None else part1 + chunk
    part = part0 if part1 is None else part0 + part1
    s_row = jnp.sum(part, axis=-1, keepdims=True)
    lse_ref[...] = jnp.log(s_row - jnp.exp2(sd_ref[...]))


def _round_up(x, mult):
    return (x + mult - 1) // mult * mult


def _pick_block(total, candidates):
    for c in candidates:
        if c <= total and total % c == 0:
            return c
    return total


def kernel(z_i, z_j, temperature=0.5):
    assert z_i.shape == z_j.shape and z_i.ndim == 2
    n, d = z_i.shape
    m = 2 * n
    inv_t = 1.0 / float(temperature)
    scale2 = math.sqrt(inv_t * _LOG2E)

    d_pad = max(128, _round_up(d, 128))
    if d_pad != d:
        z_i = jnp.pad(z_i, ((0, 0), (0, d_pad - d)))
        z_j = jnp.pad(z_j, ((0, 0), (0, d_pad - d)))

    bn = _pick_block(n, (256, 128, 64, 32, 16, 8))

    reps, kti, ktj, pos, sd = pl.pallas_call(
        functools.partial(_prep_kernel, scale2=scale2, inv_t=inv_t),
        grid=(n // bn,),
        in_specs=[pl.BlockSpec((bn, d_pad), lambda i: (i, 0)),
                  pl.BlockSpec((bn, d_pad), lambda i: (i, 0))],
        out_specs=(pl.BlockSpec((2, bn, d_pad), lambda i: (0, i, 0)),
                   pl.BlockSpec((d_pad, bn), lambda i: (0, i)),
                   pl.BlockSpec((d_pad, bn), lambda i: (0, i)),
                   pl.BlockSpec((bn, 1), lambda i: (i, 0)),
                   pl.BlockSpec((2, bn, 1), lambda i: (0, i, 0))),
        out_shape=(jax.ShapeDtypeStruct((2, n, d_pad), jnp.bfloat16),
                   jax.ShapeDtypeStruct((d_pad, n), jnp.bfloat16),
                   jax.ShapeDtypeStruct((d_pad, n), jnp.bfloat16),
                   jax.ShapeDtypeStruct((n, 1), jnp.float32),
                   jax.ShapeDtypeStruct((2, n, 1), jnp.float32)),
        compiler_params=pltpu.CompilerParams(
            dimension_semantics=("parallel",),
            vmem_limit_bytes=48 * 1024 * 1024),
    )(z_i, z_j)

    q = reps.reshape(m, d_pad)
    sd_m = sd.reshape(m, 1)

    bq = _pick_block(m, (2048, 1024, 512, 256, 128, 64, 32, 16, 8))
    sub = min(n, 1024)
    acc_w = 128 if sub % 128 == 0 else sub

    est2 = (2 * m * d_pad * 2
            + 2 * bq * d_pad * 2
            + 8 * bq * sub * 4)
    cost = pl.CostEstimate(flops=2 * m * m * d_pad,
                           transcendentals=m * m,
                           bytes_accessed=2 * m * d_pad * 2 + m * 4)

    lse = pl.pallas_call(
        functools.partial(_lse_kernel, sub=sub, acc_w=acc_w),
        grid=(m // bq,),
        in_specs=[pl.BlockSpec((bq, d_pad), lambda i: (i, 0)),
                  pl.BlockSpec((d_pad, n), lambda i: (0, 0)),
                  pl.BlockSpec((d_pad, n), lambda i: (0, 0)),
                  pl.BlockSpec((bq, 1), lambda i: (i, 0))],
        out_specs=pl.BlockSpec((bq, 1), lambda i: (i, 0)),
        out_shape=jax.ShapeDtypeStruct((m, 1), jnp.float32),
        compiler_params=pltpu.CompilerParams(
            dimension_semantics=("arbitrary",),
            vmem_limit_bytes=min(64 * 1024 * 1024,
                                 max(32 * 1024 * 1024, 2 * est2))),
        cost_estimate=cost,
    )(q, kti, ktj, sd_m)

    return (jnp.sum(lse) - 2.0 * jnp.sum(pos)) / jnp.float32(m)
